# baseline (speedup 1.0000x reference)
import functools

import jax
import jax.numpy as jnp
from jax import lax
from jax.experimental import pallas as pl
from jax.experimental.pallas import tpu as pltpu
from jax.experimental.pallas import tpu_sc as plsc

BATCH = 16384
RANK = 64
PAIR = 128
NUM_CORES = 2
NUM_SUBCORES = 16
NUM_WORKERS = NUM_CORES * NUM_SUBCORES
B_PER_W = BATCH // NUM_WORKERS
PASS_ROWS = 256
LANES = 16

BLK = 32768
QBLK = BLK // 4
BLK_SHIFT = 15
Q_SHIFT = 13
Q_MASK = QBLK - 1

NUM_U = 100001
NUM_V = 1000001

_HI_MASK = -65536


def _pack_body(src_ref, out_ref):
    x = src_ref[...]
    xi = lax.bitcast_convert_type(x, jnp.int32)
    xr = xi + 32768
    x3 = xr.reshape(32, 2, BLK)
    lo = (x3[:, 0, :] >> 16) & 65535
    hi = x3[:, 1, :] & _HI_MASK
    w = lax.bitcast_convert_type(lo | hi, jnp.float32)
    y = jnp.concatenate(
        [w[:, :QBLK], w[:, QBLK:2 * QBLK],
         w[:, 2 * QBLK:3 * QBLK], w[:, 3 * QBLK:]], axis=0)
    out_ref[...] = y.T


def _tc_pack(tableT):
    k, n = tableT.shape
    grid = (n + BLK - 1) // BLK
    return pl.pallas_call(
        _pack_body,
        grid=(grid,),
        in_specs=[pl.BlockSpec((k, BLK), lambda g: (0, g))],
        out_specs=pl.BlockSpec((QBLK, PAIR), lambda g: (g, 0)),
        out_shape=jax.ShapeDtypeStruct((grid * QBLK, PAIR), jnp.float32),
    )(tableT)


def _sc_body(users_hbm, items_hbm, pred_hbm, u2_hbm, v2_hbm, out_hbm,
             uidxA, uidxB, iidxA, iidxB, ucol, icol, urows, vrows,
             pvec, outv, accv, sem_u, sem_v):
    wid = lax.axis_index("s") * NUM_CORES + lax.axis_index("c")
    base = wid * B_PER_W

    pltpu.sync_copy(users_hbm.at[pl.ds(base, PASS_ROWS)], uidxA)
    pltpu.sync_copy(users_hbm.at[pl.ds(base + PASS_ROWS, PASS_ROWS)], uidxB)
    pltpu.sync_copy(items_hbm.at[pl.ds(base, PASS_ROWS)], iidxA)
    pltpu.sync_copy(items_hbm.at[pl.ds(base + PASS_ROWS, PASS_ROWS)], iidxB)
    pltpu.sync_copy(pred_hbm, pvec)

    def make_fix(idx_ref, col_ref, col_off):
        def fix(i, carry):
            r = idx_ref[pl.ds(i * LANES, LANES)]
            slot = ((r >> BLK_SHIFT) << Q_SHIFT) + (r & Q_MASK)
            quarter = (r >> Q_SHIFT) & 3
            idx_ref[pl.ds(i * LANES, LANES)] = slot
            col_ref[pl.ds(col_off + i * LANES, LANES)] = quarter * 32
            return carry
        return fix

    n_fix = PASS_ROWS // LANES
    lax.fori_loop(0, n_fix, make_fix(uidxA, ucol, 0), 0)
    lax.fori_loop(0, n_fix, make_fix(uidxB, ucol, PASS_ROWS), 0)
    lax.fori_loop(0, n_fix, make_fix(iidxA, icol, 0), 0)
    lax.fori_loop(0, n_fix, make_fix(iidxB, icol, PASS_ROWS), 0)

    pe = [pvec[pl.ds(0, LANES)], pvec[pl.ds(2 * LANES, LANES)]]
    po = [pvec[pl.ds(LANES, LANES)], pvec[pl.ds(3 * LANES, LANES)]]

    lane_ids = lax.iota(jnp.int32, LANES)
    col_base = lane_ids * LANES
    wchunks = [lane_ids, lane_ids + LANES]

    def unpack(wv):
        wi = plsc.bitcast(wv, jnp.int32)
        ev = plsc.bitcast(wi << 16, jnp.float32)
        od = plsc.bitcast(wi & _HI_MASK, jnp.float32)
        return ev, od

    def do_pass(pbase, uref, iref):
        cu = pltpu.async_copy(u2_hbm.at[uref], urows, sem_u)
        cv = pltpu.async_copy(v2_hbm.at[iref], vrows, sem_v)
        cu.wait()
        cv.wait()

        def group(g, carry):
            b0 = g * LANES
            for j in range(LANES):
                b = b0 + j
                bsplat = jnp.full((LANES,), b, jnp.int32)
                uco = plsc.load_gather(ucol, [bsplat + pbase])
                ico = plsc.load_gather(icol, [bsplat + pbase])
                acc = jnp.zeros((LANES,), jnp.float32)
                for c in range(2):
                    wu = plsc.load_gather(urows, [bsplat, uco + wchunks[c]])
                    wv = plsc.load_gather(vrows, [bsplat, ico + wchunks[c]])
                    ue, uo = unpack(wu)
                    ve, vo = unpack(wv)
                    acc += ue * ve * pe[c]
                    acc += uo * vo * po[c]
                accv[pl.ds(j * LANES, LANES)] = acc
            vec = plsc.load_gather(accv, [col_base])
            for k in range(1, LANES):
                vec += plsc.load_gather(accv, [col_base + k])
            outv[pl.ds(pbase + b0, LANES)] = vec
            return carry

        lax.fori_loop(0, PASS_ROWS // LANES, group, 0)

    do_pass(0, uidxA, iidxA)
    do_pass(PASS_ROWS, uidxB, iidxB)

    pltpu.sync_copy(outv, out_hbm.at[pl.ds(base, B_PER_W)])


@functools.partial(
    pl.kernel,
    mesh=plsc.VectorSubcoreMesh(core_axis_name="c", subcore_axis_name="s"),
    out_type=jax.ShapeDtypeStruct((BATCH,), jnp.float32),
    compiler_params=pltpu.CompilerParams(
        needs_layout_passes=False, use_tc_tiling_on_sc=True),
    scratch_types=[
        pltpu.VMEM((PASS_ROWS,), jnp.int32),
        pltpu.VMEM((PASS_ROWS,), jnp.int32),
        pltpu.VMEM((PASS_ROWS,), jnp.int32),
        pltpu.VMEM((PASS_ROWS,), jnp.int32),
        pltpu.VMEM((B_PER_W,), jnp.int32),
        pltpu.VMEM((B_PER_W,), jnp.int32),
        pltpu.VMEM((PASS_ROWS, PAIR), jnp.float32),
        pltpu.VMEM((PASS_ROWS, PAIR), jnp.float32),
        pltpu.VMEM((RANK,), jnp.float32),
        pltpu.VMEM((B_PER_W,), jnp.float32),
        pltpu.VMEM((LANES * LANES,), jnp.float32),
        pltpu.SemaphoreType.DMA,
        pltpu.SemaphoreType.DMA,
    ],
)
def _sc_kernel(users_hbm, items_hbm, pred_hbm, u2_hbm, v2_hbm, out_hbm,
               uidxA, uidxB, iidxA, iidxB, ucol, icol, urows, vrows,
               pvec, outv, accv, sem_u, sem_v):
    _sc_body(users_hbm, items_hbm, pred_hbm, u2_hbm, v2_hbm, out_hbm,
             uidxA, uidxB, iidxA, iidxB, ucol, icol, urows, vrows,
             pvec, outv, accv, sem_u, sem_v)


def kernel(users, items, embed_U, embed_V, predict_layer):
    pred = predict_layer.reshape(RANK)
    pred_w = jnp.concatenate(
        [pred[0:32:2], pred[1:32:2], pred[32:64:2], pred[33:64:2]])
    u2 = _tc_pack(embed_U.T)
    v2 = _tc_pack(embed_V.T)
    return _sc_kernel(users, items, pred_w, u2, v2)


# speedup vs baseline: 1.3763x; 1.3763x over previous
import functools

import jax
import jax.numpy as jnp
from jax import lax
from jax.experimental import pallas as pl
from jax.experimental.pallas import tpu as pltpu
from jax.experimental.pallas import tpu_sc as plsc

BATCH = 16384
RANK = 64
PAIR = 128
NUM_CORES = 2
NUM_SUBCORES = 16
NUM_WORKERS = NUM_CORES * NUM_SUBCORES
B_PER_W = BATCH // NUM_WORKERS
PASS_ROWS = 256
LANES = 16

BLK = 32768
QBLK = BLK // 4
BLK_SHIFT = 15
Q_SHIFT = 13
Q_MASK = QBLK - 1

NUM_U = 100001
NUM_V = 1000001

_HI_MASK = -65536


def _pack_body(src_ref, out_ref):
    x = src_ref[...]
    xi = lax.bitcast_convert_type(x, jnp.int32)
    lo = (xi[:32, :] >> 16) & 65535
    hi = xi[32:, :] & _HI_MASK
    w = lax.bitcast_convert_type(lo | hi, jnp.float32)
    y = jnp.concatenate(
        [w[:, :QBLK], w[:, QBLK:2 * QBLK],
         w[:, 2 * QBLK:3 * QBLK], w[:, 3 * QBLK:]], axis=0)
    out_ref[...] = y.T


def _tc_pack(tableT):
    k, n = tableT.shape
    grid = (n + BLK - 1) // BLK
    return pl.pallas_call(
        _pack_body,
        grid=(grid,),
        in_specs=[pl.BlockSpec((k, BLK), lambda g: (0, g))],
        out_specs=pl.BlockSpec((QBLK, PAIR), lambda g: (g, 0)),
        out_shape=jax.ShapeDtypeStruct((grid * QBLK, PAIR), jnp.float32),
    )(tableT)


def _sc_body(users_hbm, items_hbm, pred_hbm, u2_hbm, v2_hbm, out_hbm,
             uidxA, uidxB, iidxA, iidxB, ucol, icol, urows, vrows,
             pvec, outv, accv, sem_u, sem_v):
    wid = lax.axis_index("s") * NUM_CORES + lax.axis_index("c")
    base = wid * B_PER_W

    pltpu.sync_copy(users_hbm.at[pl.ds(base, PASS_ROWS)], uidxA)
    pltpu.sync_copy(users_hbm.at[pl.ds(base + PASS_ROWS, PASS_ROWS)], uidxB)
    pltpu.sync_copy(items_hbm.at[pl.ds(base, PASS_ROWS)], iidxA)
    pltpu.sync_copy(items_hbm.at[pl.ds(base + PASS_ROWS, PASS_ROWS)], iidxB)
    pltpu.sync_copy(pred_hbm, pvec)

    def make_fix(idx_ref, col_ref, col_off):
        def fix(i, carry):
            r = idx_ref[pl.ds(i * LANES, LANES)]
            slot = ((r >> BLK_SHIFT) << Q_SHIFT) + (r & Q_MASK)
            quarter = (r >> Q_SHIFT) & 3
            idx_ref[pl.ds(i * LANES, LANES)] = slot
            col_ref[pl.ds(col_off + i * LANES, LANES)] = quarter * 32
            return carry
        return fix

    n_fix = PASS_ROWS // LANES
    lax.fori_loop(0, n_fix, make_fix(uidxA, ucol, 0), 0)
    lax.fori_loop(0, n_fix, make_fix(uidxB, ucol, PASS_ROWS), 0)
    lax.fori_loop(0, n_fix, make_fix(iidxA, icol, 0), 0)
    lax.fori_loop(0, n_fix, make_fix(iidxB, icol, PASS_ROWS), 0)

    pe = [pvec[pl.ds(0, LANES)], pvec[pl.ds(LANES, LANES)]]
    po = [pvec[pl.ds(2 * LANES, LANES)], pvec[pl.ds(3 * LANES, LANES)]]

    lane_ids = lax.iota(jnp.int32, LANES)
    col_base = lane_ids * LANES
    wchunks = [lane_ids, lane_ids + LANES]

    def unpack(wv):
        wi = plsc.bitcast(wv, jnp.int32)
        ev = plsc.bitcast(wi << 16, jnp.float32)
        od = plsc.bitcast(wi & _HI_MASK, jnp.float32)
        return ev, od

    def do_pass(pbase, uref, iref):
        cu = pltpu.async_copy(u2_hbm.at[uref], urows, sem_u)
        cv = pltpu.async_copy(v2_hbm.at[iref], vrows, sem_v)
        cu.wait()
        cv.wait()

        def group(g, carry):
            b0 = g * LANES
            for j in range(LANES):
                b = b0 + j
                bsplat = jnp.full((LANES,), b, jnp.int32)
                uco = plsc.load_gather(ucol, [bsplat + pbase])
                ico = plsc.load_gather(icol, [bsplat + pbase])
                acc = jnp.zeros((LANES,), jnp.float32)
                for c in range(2):
                    wu = plsc.load_gather(urows, [bsplat, uco + wchunks[c]])
                    wv = plsc.load_gather(vrows, [bsplat, ico + wchunks[c]])
                    ue, uo = unpack(wu)
                    ve, vo = unpack(wv)
                    acc += ue * ve * pe[c]
                    acc += uo * vo * po[c]
                accv[pl.ds(j * LANES, LANES)] = acc
            vec = plsc.load_gather(accv, [col_base])
            for k in range(1, LANES):
                vec += plsc.load_gather(accv, [col_base + k])
            outv[pl.ds(pbase + b0, LANES)] = vec
            return carry

        lax.fori_loop(0, PASS_ROWS // LANES, group, 0)

    do_pass(0, uidxA, iidxA)
    do_pass(PASS_ROWS, uidxB, iidxB)

    pltpu.sync_copy(outv, out_hbm.at[pl.ds(base, B_PER_W)])


@functools.partial(
    pl.kernel,
    mesh=plsc.VectorSubcoreMesh(core_axis_name="c", subcore_axis_name="s"),
    out_type=jax.ShapeDtypeStruct((BATCH,), jnp.float32),
    compiler_params=pltpu.CompilerParams(
        needs_layout_passes=False, use_tc_tiling_on_sc=True),
    scratch_types=[
        pltpu.VMEM((PASS_ROWS,), jnp.int32),
        pltpu.VMEM((PASS_ROWS,), jnp.int32),
        pltpu.VMEM((PASS_ROWS,), jnp.int32),
        pltpu.VMEM((PASS_ROWS,), jnp.int32),
        pltpu.VMEM((B_PER_W,), jnp.int32),
        pltpu.VMEM((B_PER_W,), jnp.int32),
        pltpu.VMEM((PASS_ROWS, PAIR), jnp.float32),
        pltpu.VMEM((PASS_ROWS, PAIR), jnp.float32),
        pltpu.VMEM((RANK,), jnp.float32),
        pltpu.VMEM((B_PER_W,), jnp.float32),
        pltpu.VMEM((LANES * LANES,), jnp.float32),
        pltpu.SemaphoreType.DMA,
        pltpu.SemaphoreType.DMA,
    ],
)
def _sc_kernel(users_hbm, items_hbm, pred_hbm, u2_hbm, v2_hbm, out_hbm,
               uidxA, uidxB, iidxA, iidxB, ucol, icol, urows, vrows,
               pvec, outv, accv, sem_u, sem_v):
    _sc_body(users_hbm, items_hbm, pred_hbm, u2_hbm, v2_hbm, out_hbm,
             uidxA, uidxB, iidxA, iidxB, ucol, icol, urows, vrows,
             pvec, outv, accv, sem_u, sem_v)


def kernel(users, items, embed_U, embed_V, predict_layer):
    pred = predict_layer.reshape(RANK)
    u2 = _tc_pack(embed_U.T)
    v2 = _tc_pack(embed_V.T)
    return _sc_kernel(users, items, pred, u2, v2)


# speedup vs baseline: 1.3788x; 1.0018x over previous
import functools

import jax
import jax.numpy as jnp
from jax import lax
from jax.experimental import pallas as pl
from jax.experimental.pallas import tpu as pltpu
from jax.experimental.pallas import tpu_sc as plsc

BATCH = 16384
RANK = 64
PAIR = 128
NUM_CORES = 2
NUM_SUBCORES = 16
NUM_WORKERS = NUM_CORES * NUM_SUBCORES
B_PER_W = BATCH // NUM_WORKERS
PASS_ROWS = 256
LANES = 16

BLK = 32768
QBLK = BLK // 4
BLK_SHIFT = 15
Q_SHIFT = 13
Q_MASK = QBLK - 1

NUM_U = 100001
NUM_V = 1000001

_HI_MASK = -65536


def _pack_body(src_ref, out_ref):
    x = src_ref[...]
    xi = lax.bitcast_convert_type(x, jnp.int32) + 32768
    lo = (xi[:32, :] >> 16) & 65535
    hi = xi[32:, :] & _HI_MASK
    w = lax.bitcast_convert_type(lo | hi, jnp.float32)
    y = jnp.concatenate(
        [w[:, :QBLK], w[:, QBLK:2 * QBLK],
         w[:, 2 * QBLK:3 * QBLK], w[:, 3 * QBLK:]], axis=0)
    out_ref[...] = y.T


def _tc_pack(tableT):
    k, n = tableT.shape
    grid = (n + BLK - 1) // BLK
    return pl.pallas_call(
        _pack_body,
        grid=(grid,),
        in_specs=[pl.BlockSpec((k, BLK), lambda g: (0, g))],
        out_specs=pl.BlockSpec((QBLK, PAIR), lambda g: (g, 0)),
        out_shape=jax.ShapeDtypeStruct((grid * QBLK, PAIR), jnp.float32),
    )(tableT)


def _sc_body(users_hbm, items_hbm, pred_hbm, u2_hbm, v2_hbm, out_hbm,
             uidxA, uidxB, iidxA, iidxB, ucol, icol, urows, vrows,
             pvec, outv, accv, sem_u, sem_v):
    wid = lax.axis_index("s") * NUM_CORES + lax.axis_index("c")
    base = wid * B_PER_W

    pltpu.sync_copy(users_hbm.at[pl.ds(base, PASS_ROWS)], uidxA)
    pltpu.sync_copy(users_hbm.at[pl.ds(base + PASS_ROWS, PASS_ROWS)], uidxB)
    pltpu.sync_copy(items_hbm.at[pl.ds(base, PASS_ROWS)], iidxA)
    pltpu.sync_copy(items_hbm.at[pl.ds(base + PASS_ROWS, PASS_ROWS)], iidxB)
    pltpu.sync_copy(pred_hbm, pvec)

    def make_fix(idx_ref, col_ref, col_off):
        def fix(i, carry):
            r = idx_ref[pl.ds(i * LANES, LANES)]
            slot = ((r >> BLK_SHIFT) << Q_SHIFT) + (r & Q_MASK)
            quarter = (r >> Q_SHIFT) & 3
            idx_ref[pl.ds(i * LANES, LANES)] = slot
            col_ref[pl.ds(col_off + i * LANES, LANES)] = quarter * 32
            return carry
        return fix

    n_fix = PASS_ROWS // LANES
    lax.fori_loop(0, n_fix, make_fix(uidxA, ucol, 0), 0)
    lax.fori_loop(0, n_fix, make_fix(uidxB, ucol, PASS_ROWS), 0)
    lax.fori_loop(0, n_fix, make_fix(iidxA, icol, 0), 0)
    lax.fori_loop(0, n_fix, make_fix(iidxB, icol, PASS_ROWS), 0)

    pe = [pvec[pl.ds(0, LANES)], pvec[pl.ds(LANES, LANES)]]
    po = [pvec[pl.ds(2 * LANES, LANES)], pvec[pl.ds(3 * LANES, LANES)]]

    lane_ids = lax.iota(jnp.int32, LANES)
    col_base = lane_ids * LANES
    wchunks = [lane_ids, lane_ids + LANES]

    def unpack(wv):
        wi = plsc.bitcast(wv, jnp.int32)
        ev = plsc.bitcast(wi << 16, jnp.float32)
        od = plsc.bitcast(wi & _HI_MASK, jnp.float32)
        return ev, od

    def do_pass(pbase, uref, iref):
        cu = pltpu.async_copy(u2_hbm.at[uref], urows, sem_u)
        cv = pltpu.async_copy(v2_hbm.at[iref], vrows, sem_v)
        cu.wait()
        cv.wait()

        def group(g, carry):
            b0 = g * LANES
            for j in range(LANES):
                b = b0 + j
                bsplat = jnp.full((LANES,), b, jnp.int32)
                uco = plsc.load_gather(ucol, [bsplat + pbase])
                ico = plsc.load_gather(icol, [bsplat + pbase])
                acc = jnp.zeros((LANES,), jnp.float32)
                for c in range(2):
                    wu = plsc.load_gather(urows, [bsplat, uco + wchunks[c]])
                    wv = plsc.load_gather(vrows, [bsplat, ico + wchunks[c]])
                    ue, uo = unpack(wu)
                    ve, vo = unpack(wv)
                    acc += ue * ve * pe[c]
                    acc += uo * vo * po[c]
                accv[pl.ds(j * LANES, LANES)] = acc
            vec = plsc.load_gather(accv, [col_base])
            for k in range(1, LANES):
                vec += plsc.load_gather(accv, [col_base + k])
            outv[pl.ds(pbase + b0, LANES)] = vec
            return carry

        lax.fori_loop(0, PASS_ROWS // LANES, group, 0)

    do_pass(0, uidxA, iidxA)
    do_pass(PASS_ROWS, uidxB, iidxB)

    pltpu.sync_copy(outv, out_hbm.at[pl.ds(base, B_PER_W)])


@functools.partial(
    pl.kernel,
    mesh=plsc.VectorSubcoreMesh(core_axis_name="c", subcore_axis_name="s"),
    out_type=jax.ShapeDtypeStruct((BATCH,), jnp.float32),
    compiler_params=pltpu.CompilerParams(
        needs_layout_passes=False, use_tc_tiling_on_sc=True),
    scratch_types=[
        pltpu.VMEM((PASS_ROWS,), jnp.int32),
        pltpu.VMEM((PASS_ROWS,), jnp.int32),
        pltpu.VMEM((PASS_ROWS,), jnp.int32),
        pltpu.VMEM((PASS_ROWS,), jnp.int32),
        pltpu.VMEM((B_PER_W,), jnp.int32),
        pltpu.VMEM((B_PER_W,), jnp.int32),
        pltpu.VMEM((PASS_ROWS, PAIR), jnp.float32),
        pltpu.VMEM((PASS_ROWS, PAIR), jnp.float32),
        pltpu.VMEM((RANK,), jnp.float32),
        pltpu.VMEM((B_PER_W,), jnp.float32),
        pltpu.VMEM((LANES * LANES,), jnp.float32),
        pltpu.SemaphoreType.DMA,
        pltpu.SemaphoreType.DMA,
    ],
)
def _sc_kernel(users_hbm, items_hbm, pred_hbm, u2_hbm, v2_hbm, out_hbm,
               uidxA, uidxB, iidxA, iidxB, ucol, icol, urows, vrows,
               pvec, outv, accv, sem_u, sem_v):
    _sc_body(users_hbm, items_hbm, pred_hbm, u2_hbm, v2_hbm, out_hbm,
             uidxA, uidxB, iidxA, iidxB, ucol, icol, urows, vrows,
             pvec, outv, accv, sem_u, sem_v)


def kernel(users, items, embed_U, embed_V, predict_layer):
    pred = predict_layer.reshape(RANK)
    u2 = _tc_pack(embed_U.T)
    v2 = _tc_pack(embed_V.T)
    return _sc_kernel(users, items, pred, u2, v2)
